# baseline (speedup 1.0000x reference)
import functools

import jax
import jax.numpy as jnp
from jax import lax
from jax.experimental import pallas as pl
from jax.experimental.pallas import tpu as pltpu
from jax.experimental.pallas import tpu_sc as plsc

_B, _C, _N, _L = 2, 3, 4, 64
_BCN = _B * _C * _N
_SLAB = 8
_NSLAB = _L // _SLAB
_SLAB_WORDS = _SLAB * _L * _L
_BOX_WORDS = _L * _L * _L
_F32 = jnp.float32


def _sc_body(l_hbm, m_hbm, o_dp, o_dm, o_hp, o_hm, o_wp, o_wm,
             lbuf, mbuf, s_dp, s_dm, s_hp, s_hm, s_wp, s_wm):
    wid = lax.axis_index("s") * 2 + lax.axis_index("c")

    @pl.when(wid < _BCN)
    def _():
        bcn = wid
        bc = bcn // _N
        zero = jnp.zeros((16,), _F32)

        for dc in range(4):
            s_dp[pl.ds(dc * 16, 16)] = zero
            s_dm[pl.ds(dc * 16, 16)] = zero

        def zero_h(h, _):
            s_hp[pl.ds(h * 16, 16)] = zero
            s_hm[pl.ds(h * 16, 16)] = zero
            return 0

        lax.fori_loop(0, _L, zero_h, 0)

        def slab_body(slab, _):
            lbase = pl.multiple_of(bc * _BOX_WORDS + slab * _SLAB_WORDS, 8)
            mbase = pl.multiple_of(bcn * _BOX_WORDS + slab * _SLAB_WORDS, 8)
            pltpu.sync_copy(l_hbm.at[pl.ds(lbase, _SLAB_WORDS)], lbuf)
            pltpu.sync_copy(m_hbm.at[pl.ds(mbase, _SLAB_WORDS)], mbuf)

            def h_body(h, carry):
                acc = list(carry)
                hacc = [zero] * 4
                mhacc = [zero] * 4
                hbase = h * _L
                for w in range(_SLAB):
                    for dc in range(4):
                        off = hbase + (w * _L * _L + dc * 16)
                        lv = lbuf[pl.ds(off, 16)]
                        mv = mbuf[pl.ds(off, 16)]
                        pv = lv * mv
                        acc[dc] = acc[dc] + pv
                        acc[4 + dc] = acc[4 + dc] + mv
                        hacc[dc] = hacc[dc] + pv
                        mhacc[dc] = mhacc[dc] + mv
                        acc[8 + w] = acc[8 + w] + pv
                        acc[16 + w] = acc[16 + w] + mv
                hsum = (hacc[0] + hacc[1]) + (hacc[2] + hacc[3])
                mhsum = (mhacc[0] + mhacc[1]) + (mhacc[2] + mhacc[3])
                plsc.addupdate(s_hp.at[pl.ds(h * 16, 16)], hsum)
                plsc.addupdate(s_hm.at[pl.ds(h * 16, 16)], mhsum)
                return tuple(acc)

            acc = lax.fori_loop(0, _L, h_body, (zero,) * 24)
            for dc in range(4):
                plsc.addupdate(s_dp.at[pl.ds(dc * 16, 16)], acc[dc])
                plsc.addupdate(s_dm.at[pl.ds(dc * 16, 16)], acc[4 + dc])
            for w in range(_SLAB):
                woff = (slab * _SLAB + w) * 16
                s_wp[pl.ds(woff, 16)] = acc[8 + w]
                s_wm[pl.ds(woff, 16)] = acc[16 + w]
            return 0

        lax.fori_loop(0, _NSLAB, slab_body, 0)

        pltpu.sync_copy(s_dp, o_dp.at[bcn])
        pltpu.sync_copy(s_dm, o_dm.at[bcn])
        pltpu.sync_copy(s_hp, o_hp.at[bcn])
        pltpu.sync_copy(s_hm, o_hm.at[bcn])
        pltpu.sync_copy(s_wp, o_wp.at[bcn])
        pltpu.sync_copy(s_wm, o_wm.at[bcn])


_sc_profiles = functools.partial(
    pl.kernel,
    out_type=(
        jax.ShapeDtypeStruct((_BCN, 64), _F32),
        jax.ShapeDtypeStruct((_BCN, 64), _F32),
        jax.ShapeDtypeStruct((_BCN, 1024), _F32),
        jax.ShapeDtypeStruct((_BCN, 1024), _F32),
        jax.ShapeDtypeStruct((_BCN, 1024), _F32),
        jax.ShapeDtypeStruct((_BCN, 1024), _F32),
    ),
    mesh=plsc.VectorSubcoreMesh(core_axis_name="c", subcore_axis_name="s"),
    scratch_types=[
        pltpu.VMEM((_SLAB_WORDS,), _F32),
        pltpu.VMEM((_SLAB_WORDS,), _F32),
        pltpu.VMEM((64,), _F32),
        pltpu.VMEM((64,), _F32),
        pltpu.VMEM((1024,), _F32),
        pltpu.VMEM((1024,), _F32),
        pltpu.VMEM((1024,), _F32),
        pltpu.VMEM((1024,), _F32),
    ],
)(_sc_body)


def _axis_err(s, mcnt):
    m = (mcnt > 0).astype(_F32)
    cnt = m.sum(-1)
    valid = cnt > 0
    mean = jnp.where(valid, (s * m).sum(-1) / jnp.maximum(cnt, 1.0), 0.0)
    return jnp.maximum(jnp.where(valid, 1.0 - mean, 0.0), 0.0)


def _epi_body(dp, dm, hp, hm, wp, wm, out):
    err_d = _axis_err(dp[...], dm[...])
    err_h = _axis_err(hp[...].sum(-1), hm[...].sum(-1))
    err_w = _axis_err(wp[...].sum(-1), wm[...].sum(-1))
    tot = (err_d + err_h + err_w).sum(-1) * 8.0
    out[0, 0] = jnp.sum(tot * tot)


def kernel(logits, box_masks):
    lflat = jnp.reshape(logits, (-1,))
    mflat = jnp.reshape(box_masks, (-1,))
    dp, dm, hp, hm, wp, wm = _sc_profiles(lflat, mflat)
    loss = pl.pallas_call(
        _epi_body,
        out_shape=jax.ShapeDtypeStruct((1, 1), _F32),
        out_specs=pl.BlockSpec(memory_space=pltpu.SMEM),
    )(
        dp.reshape(_BCN, 8, 8),
        dm.reshape(_BCN, 8, 8),
        hp.reshape(_BCN, 8, 8, 16),
        hm.reshape(_BCN, 8, 8, 16),
        wp.reshape(_BCN, 8, 8, 16),
        wm.reshape(_BCN, 8, 8, 16),
    )
    return loss[0, 0]


# speedup vs baseline: 1.2553x; 1.2553x over previous
import functools

import jax
import jax.numpy as jnp
from jax import lax
from jax.experimental import pallas as pl
from jax.experimental.pallas import tpu as pltpu
from jax.experimental.pallas import tpu_sc as plsc

_B, _C, _N, _L = 2, 3, 4, 64
_BCN = _B * _C * _N
_SLAB = 4
_NSLAB = _L // _SLAB
_F32 = jnp.float32


def _sc_body(l_hbm, m_hbm, o_dp, o_hp, o_wp, lbuf, mbuf, s_dp, s_hp, s_wp):
    wid = lax.axis_index("s") * 2 + lax.axis_index("c")

    @pl.when(wid < _BCN)
    def _():
        bcn = wid
        b = bcn // (_C * _N)
        c = (bcn // _N) % _C
        n = bcn % _N
        zero = jnp.zeros((16,), _F32)

        for dc in range(4):
            s_dp[pl.ds(dc * 16, 16)] = zero

        def zero_h(h, _):
            s_hp[pl.ds(h * 16, 16)] = zero
            return 0

        lax.fori_loop(0, _L, zero_h, 0)

        def slab_body(slab, _):
            w0 = pl.multiple_of(slab * _SLAB, _SLAB)
            pltpu.sync_copy(l_hbm.at[b, c, pl.ds(w0, _SLAB)], lbuf)
            pltpu.sync_copy(m_hbm.at[b, c, n, pl.ds(w0, _SLAB)], mbuf)

            def h_body(h, carry):
                acc = list(carry)
                hacc = [zero] * 4
                for w in range(_SLAB):
                    for dc in range(4):
                        lv = lbuf[w, h, pl.ds(dc * 16, 16)]
                        mv = mbuf[w, h, pl.ds(dc * 16, 16)]
                        pv = lv * mv
                        acc[dc] = acc[dc] + pv
                        hacc[dc] = hacc[dc] + pv
                        acc[4 + w] = acc[4 + w] + pv
                hsum = (hacc[0] + hacc[1]) + (hacc[2] + hacc[3])
                plsc.addupdate(s_hp.at[pl.ds(h * 16, 16)], hsum)
                return tuple(acc)

            acc = lax.fori_loop(0, _L, h_body, (zero,) * (4 + _SLAB))
            for dc in range(4):
                plsc.addupdate(s_dp.at[pl.ds(dc * 16, 16)], acc[dc])
            for w in range(_SLAB):
                s_wp[pl.ds((slab * _SLAB + w) * 16, 16)] = acc[4 + w]
            return 0

        lax.fori_loop(0, _NSLAB, slab_body, 0)

        pltpu.sync_copy(s_dp, o_dp.at[bcn])
        pltpu.sync_copy(s_hp, o_hp.at[bcn])
        pltpu.sync_copy(s_wp, o_wp.at[bcn])


_sc_profiles = functools.partial(
    pl.kernel,
    out_type=(
        jax.ShapeDtypeStruct((_BCN, 64), _F32),
        jax.ShapeDtypeStruct((_BCN, 1024), _F32),
        jax.ShapeDtypeStruct((_BCN, 1024), _F32),
    ),
    mesh=plsc.VectorSubcoreMesh(core_axis_name="c", subcore_axis_name="s"),
    scratch_types=[
        pltpu.VMEM((_SLAB, _L, _L), _F32),
        pltpu.VMEM((_SLAB, _L, _L), _F32),
        pltpu.VMEM((64,), _F32),
        pltpu.VMEM((1024,), _F32),
        pltpu.VMEM((1024,), _F32),
    ],
    compiler_params=pltpu.CompilerParams(use_tc_tiling_on_sc=True),
)(_sc_body)


def _axis_err(s):
    m = (s > 0).astype(_F32)
    cnt = m.sum(-1)
    valid = cnt > 0
    mean = jnp.where(valid, s.sum(-1) / jnp.maximum(cnt, 1.0), 0.0)
    return jnp.maximum(jnp.where(valid, 1.0 - mean, 0.0), 0.0)


def _epi_body(dp, hp, wp, out):
    err_d = _axis_err(dp[...])
    err_h = _axis_err(hp[...].sum(-1))
    err_w = _axis_err(wp[...].sum(-1))
    tot = (err_d + err_h + err_w).sum(-1) * 8.0
    out[0, 0] = jnp.sum(tot * tot)


def kernel(logits, box_masks):
    dp, hp, wp = _sc_profiles(logits, box_masks)
    loss = pl.pallas_call(
        _epi_body,
        out_shape=jax.ShapeDtypeStruct((1, 1), _F32),
        out_specs=pl.BlockSpec(memory_space=pltpu.SMEM),
    )(
        dp.reshape(_BCN, 8, 8),
        hp.reshape(_BCN, 8, 8, 16),
        wp.reshape(_BCN, 8, 8, 16),
    )
    return loss[0, 0]


# speedup vs baseline: 2.5886x; 2.0621x over previous
import functools

import jax
import jax.numpy as jnp
from jax import lax
from jax.experimental import pallas as pl
from jax.experimental.pallas import tpu as pltpu
from jax.experimental.pallas import tpu_sc as plsc

_B, _C, _N, _L = 2, 3, 4, 64
_BCN = _B * _C * _N
_SCB = 8
_NQ = 4
_QW = _L // _NQ
_SLAB = 4
_NSLAB = _QW // _SLAB
_F32 = jnp.float32


def _sc_body(l_hbm, m_hbm, o_dp, o_hp, o_wp,
             lbuf, mbuf, s_dp, s_hp, s_wp):
    wid = lax.axis_index("s") * 2 + lax.axis_index("c")
    box = wid // _NQ
    quarter = wid % _NQ
    b = box // (_C * _N)
    c = (box // _N) % _C
    n = box % _N
    zero = jnp.zeros((16,), _F32)

    for dc in range(4):
        s_dp[pl.ds(dc * 16, 16)] = zero

    def zero_h(h, _):
        s_hp[pl.ds(h * 16, 16)] = zero
        return 0

    lax.fori_loop(0, _L, zero_h, 0)

    def slab_body(slab, _):
        w0 = pl.multiple_of(quarter * _QW + slab * _SLAB, _SLAB)
        pltpu.sync_copy(l_hbm.at[b, c, pl.ds(w0, _SLAB)], lbuf)
        pltpu.sync_copy(m_hbm.at[b, c, n, pl.ds(w0, _SLAB)], mbuf)

        def h_body(h, carry):
            acc = list(carry)
            hacc = [zero] * 4
            for w in range(_SLAB):
                for dc in range(4):
                    lv = lbuf[w, h, pl.ds(dc * 16, 16)]
                    mv = mbuf[w, h, pl.ds(dc * 16, 16)]
                    pv = lv * mv
                    acc[dc] = acc[dc] + pv
                    hacc[dc] = hacc[dc] + pv
                    acc[4 + w] = acc[4 + w] + pv
            hsum = (hacc[0] + hacc[1]) + (hacc[2] + hacc[3])
            plsc.addupdate(s_hp.at[pl.ds(h * 16, 16)], hsum)
            return tuple(acc)

        acc = lax.fori_loop(0, _L, h_body, (zero,) * (4 + _SLAB))
        for dc in range(4):
            plsc.addupdate(s_dp.at[pl.ds(dc * 16, 16)], acc[dc])
        for w in range(_SLAB):
            s_wp[pl.ds((slab * _SLAB + w) * 16, 16)] = acc[4 + w]
        return 0

    lax.fori_loop(0, _NSLAB, slab_body, 0)

    pltpu.sync_copy(s_dp, o_dp.at[wid])
    pltpu.sync_copy(s_hp, o_hp.at[wid])
    pltpu.sync_copy(s_wp, o_wp.at[wid])


_sc_profiles = functools.partial(
    pl.kernel,
    out_type=(
        jax.ShapeDtypeStruct((_SCB * _NQ, 64), _F32),
        jax.ShapeDtypeStruct((_SCB * _NQ, 1024), _F32),
        jax.ShapeDtypeStruct((_SCB * _NQ, _QW * 16), _F32),
    ),
    mesh=plsc.VectorSubcoreMesh(core_axis_name="c", subcore_axis_name="s"),
    scratch_types=[
        pltpu.VMEM((_SLAB, _L, _L), _F32),
        pltpu.VMEM((_SLAB, _L, _L), _F32),
        pltpu.VMEM((64,), _F32),
        pltpu.VMEM((1024,), _F32),
        pltpu.VMEM((_QW * 16,), _F32),
    ],
    compiler_params=pltpu.CompilerParams(use_tc_tiling_on_sc=True),
)(_sc_body)


_TCB = _BCN - _SCB
_TCG = _TCB // _N


def _tc_body(l_ref, m_ref, dp_ref, hp_ref, wp_ref):
    lg = l_ref[0, 0]
    for n in range(_N):
        p = lg * m_ref[0, 0, n]
        a = p.sum(axis=0)
        dp_ref[0, n] = a.sum(axis=0)
        hp_ref[0, n] = a.sum(axis=1)
        wp_ref[0, n] = p.sum(axis=(1, 2))


def _tc_index_l(g):
    return (g + _SCB // _N) // _C, (g + _SCB // _N) % _C, 0, 0, 0


def _tc_index_m(g):
    return (g + _SCB // _N) // _C, (g + _SCB // _N) % _C, 0, 0, 0, 0


_tc_profiles = functools.partial(
    pl.pallas_call,
    grid=(_TCG,),
    in_specs=[
        pl.BlockSpec((1, 1, _L, _L, _L), _tc_index_l),
        pl.BlockSpec((1, 1, _N, _L, _L, _L), _tc_index_m),
    ],
    out_specs=[
        pl.BlockSpec((1, _N, _L), lambda g: (g, 0, 0)),
        pl.BlockSpec((1, _N, _L), lambda g: (g, 0, 0)),
        pl.BlockSpec((1, _N, _L), lambda g: (g, 0, 0)),
    ],
    out_shape=[
        jax.ShapeDtypeStruct((_TCG, _N, _L), _F32),
        jax.ShapeDtypeStruct((_TCG, _N, _L), _F32),
        jax.ShapeDtypeStruct((_TCG, _N, _L), _F32),
    ],
    compiler_params=pltpu.CompilerParams(
        dimension_semantics=("arbitrary",),
    ),
)(_tc_body)


def _axis_err(s):
    m = (s > 0).astype(_F32)
    cnt = m.sum(-1)
    valid = cnt > 0
    mean = jnp.where(valid, s.sum(-1) / jnp.maximum(cnt, 1.0), 0.0)
    return jnp.maximum(jnp.where(valid, 1.0 - mean, 0.0), 0.0)


def _epi_body(sdp, shp, swp, tdp, thp, twp, out):
    e_d = _axis_err(sdp[...].sum(axis=1)).sum(-1)
    e_h = _axis_err(shp[...].sum(axis=(1, 4))).sum(-1)
    e_w = _axis_err(swp[...].sum(axis=4)).sum(axis=(1, 2))
    tot_sc = (e_d + e_h + e_w) * 8.0
    f_d = _axis_err(tdp[...]).sum(-1)
    f_h = _axis_err(thp[...]).sum(-1)
    f_w = _axis_err(twp[...]).sum(-1)
    tot_tc = (f_d + f_h + f_w) * 8.0
    out[0, 0] = jnp.sum(tot_sc * tot_sc) + jnp.sum(tot_tc * tot_tc)


def kernel(logits, box_masks):
    sdp, shp, swp = _sc_profiles(logits, box_masks)
    tdp, thp, twp = _tc_profiles(logits, box_masks)
    loss = pl.pallas_call(
        _epi_body,
        out_shape=jax.ShapeDtypeStruct((1, 1), _F32),
        out_specs=pl.BlockSpec(memory_space=pltpu.SMEM),
    )(
        sdp.reshape(_SCB, _NQ, 8, 8),
        shp.reshape(_SCB, _NQ, 8, 8, 16),
        swp.reshape(_SCB, _NQ, 2, 8, 16),
        tdp.reshape(_TCB, 8, 8),
        thp.reshape(_TCB, 8, 8),
        twp.reshape(_TCB, 8, 8),
    )
    return loss[0, 0]


# speedup vs baseline: 3.2290x; 1.2474x over previous
import functools

import jax
import jax.numpy as jnp
from jax import lax
from jax.experimental import pallas as pl
from jax.experimental.pallas import tpu as pltpu
from jax.experimental.pallas import tpu_sc as plsc

_B, _C, _N, _L = 2, 3, 4, 64
_BCN = _B * _C * _N
_SCB = 4
_NQ = 8
_QW = _L // _NQ
_SLAB = 4
_NSLAB = _QW // _SLAB
_F32 = jnp.float32


def _sc_body(l_hbm, m_hbm, o_dp, o_hp, o_wp,
             lbuf, mbuf, s_dp, s_hp, s_wp):
    wid = lax.axis_index("s") * 2 + lax.axis_index("c")
    box = wid // _NQ
    quarter = wid % _NQ
    b = box // (_C * _N)
    c = (box // _N) % _C
    n = box % _N
    zero = jnp.zeros((16,), _F32)

    for dc in range(4):
        s_dp[pl.ds(dc * 16, 16)] = zero

    def zero_h(h, _):
        s_hp[pl.ds(h * 16, 16)] = zero
        return 0

    lax.fori_loop(0, _L, zero_h, 0)

    def slab_body(slab, _):
        w0 = pl.multiple_of(quarter * _QW + slab * _SLAB, _SLAB)
        pltpu.sync_copy(l_hbm.at[b, c, pl.ds(w0, _SLAB)], lbuf)
        pltpu.sync_copy(m_hbm.at[b, c, n, pl.ds(w0, _SLAB)], mbuf)

        def h_body(h, carry):
            acc = list(carry)
            hacc = [zero] * 4
            for w in range(_SLAB):
                for dc in range(4):
                    lv = lbuf[w, h, pl.ds(dc * 16, 16)]
                    mv = mbuf[w, h, pl.ds(dc * 16, 16)]
                    pv = lv * mv
                    acc[dc] = acc[dc] + pv
                    hacc[dc] = hacc[dc] + pv
                    acc[4 + w] = acc[4 + w] + pv
            hsum = (hacc[0] + hacc[1]) + (hacc[2] + hacc[3])
            plsc.addupdate(s_hp.at[pl.ds(h * 16, 16)], hsum)
            return tuple(acc)

        acc = lax.fori_loop(0, _L, h_body, (zero,) * (4 + _SLAB))
        for dc in range(4):
            plsc.addupdate(s_dp.at[pl.ds(dc * 16, 16)], acc[dc])
        for w in range(_SLAB):
            s_wp[pl.ds((slab * _SLAB + w) * 16, 16)] = acc[4 + w]
        return 0

    lax.fori_loop(0, _NSLAB, slab_body, 0)

    pltpu.sync_copy(s_dp, o_dp.at[wid])
    pltpu.sync_copy(s_hp, o_hp.at[wid])
    pltpu.sync_copy(s_wp, o_wp.at[wid])


_sc_profiles = functools.partial(
    pl.kernel,
    out_type=(
        jax.ShapeDtypeStruct((_SCB * _NQ, 64), _F32),
        jax.ShapeDtypeStruct((_SCB * _NQ, 1024), _F32),
        jax.ShapeDtypeStruct((_SCB * _NQ, _QW * 16), _F32),
    ),
    mesh=plsc.VectorSubcoreMesh(core_axis_name="c", subcore_axis_name="s"),
    scratch_types=[
        pltpu.VMEM((_SLAB, _L, _L), _F32),
        pltpu.VMEM((_SLAB, _L, _L), _F32),
        pltpu.VMEM((64,), _F32),
        pltpu.VMEM((1024,), _F32),
        pltpu.VMEM((_QW * 16,), _F32),
    ],
    compiler_params=pltpu.CompilerParams(use_tc_tiling_on_sc=True),
)(_sc_body)


_TCB = _BCN - _SCB
_TCG = _TCB // _N


def _tc_body(l_ref, m_ref, dp_ref, hp_ref, wp_ref):
    lg = l_ref[0, 0]
    for n in range(_N):
        p = lg * m_ref[0, 0, n]
        a = p.sum(axis=0)
        dp_ref[0, n] = a.sum(axis=0)
        hp_ref[0, n] = a.sum(axis=1)
        wp_ref[0, n] = p.sum(axis=(1, 2))


def _tc_index_l(g):
    return (g + _SCB // _N) // _C, (g + _SCB // _N) % _C, 0, 0, 0


def _tc_index_m(g):
    return (g + _SCB // _N) // _C, (g + _SCB // _N) % _C, 0, 0, 0, 0


_tc_profiles = functools.partial(
    pl.pallas_call,
    grid=(_TCG,),
    in_specs=[
        pl.BlockSpec((1, 1, _L, _L, _L), _tc_index_l),
        pl.BlockSpec((1, 1, _N, _L, _L, _L), _tc_index_m),
    ],
    out_specs=[
        pl.BlockSpec((1, _N, _L), lambda g: (g, 0, 0)),
        pl.BlockSpec((1, _N, _L), lambda g: (g, 0, 0)),
        pl.BlockSpec((1, _N, _L), lambda g: (g, 0, 0)),
    ],
    out_shape=[
        jax.ShapeDtypeStruct((_TCG, _N, _L), _F32),
        jax.ShapeDtypeStruct((_TCG, _N, _L), _F32),
        jax.ShapeDtypeStruct((_TCG, _N, _L), _F32),
    ],
    compiler_params=pltpu.CompilerParams(
        dimension_semantics=("arbitrary",),
    ),
)(_tc_body)


def _axis_err(s):
    m = (s > 0).astype(_F32)
    cnt = m.sum(-1)
    valid = cnt > 0
    mean = jnp.where(valid, s.sum(-1) / jnp.maximum(cnt, 1.0), 0.0)
    return jnp.maximum(jnp.where(valid, 1.0 - mean, 0.0), 0.0)


def _lane_sel(rows, cols):
    r = lax.broadcasted_iota(jnp.int32, (rows, cols), 0)
    c = lax.broadcasted_iota(jnp.int32, (rows, cols), 1)
    return (r // 16 == c).astype(_F32)


def _win_errs(mat):
    tot = _axis_err(mat[:, 0:8])
    for win in range(1, 8):
        tot = tot + _axis_err(mat[:, win * 8:(win + 1) * 8])
    return tot


def _epi_body(sdp, shp, swp, tdp, thp, twp, out):
    hi = jax.lax.Precision.HIGHEST
    sdp3 = sdp[...].reshape(_SCB, _NQ, 64).sum(axis=1)
    shp_h = jnp.dot(shp[...].reshape(_SCB, _NQ, 1024).sum(axis=1),
                    _lane_sel(1024, 64), precision=hi)
    swp_w = jnp.dot(swp[...], _lane_sel(_QW * 16, _QW), precision=hi)
    e_w = _axis_err(swp_w).reshape(_SCB, _NQ).sum(axis=1)
    tot_sc = (_win_errs(sdp3) + _win_errs(shp_h) + e_w) * 8.0
    tdp2 = tdp[...].reshape(_TCB, 64)
    thp2 = thp[...].reshape(_TCB, 64)
    twp2 = twp[...].reshape(_TCB, 64)
    tot_tc = (_win_errs(tdp2) + _win_errs(thp2) + _win_errs(twp2)) * 8.0
    out[0, 0] = jnp.sum(tot_sc * tot_sc) + jnp.sum(tot_tc * tot_tc)


def kernel(logits, box_masks):
    sdp, shp, swp = _sc_profiles(logits, box_masks)
    tdp, thp, twp = _tc_profiles(logits, box_masks)
    loss = pl.pallas_call(
        _epi_body,
        out_shape=jax.ShapeDtypeStruct((1, 1), _F32),
        out_specs=pl.BlockSpec(memory_space=pltpu.SMEM),
    )(sdp, shp, swp, tdp, thp, twp)
    return loss[0, 0]


# speedup vs baseline: 3.2382x; 1.0028x over previous
import functools

import jax
import jax.numpy as jnp
from jax import lax
from jax.experimental import pallas as pl
from jax.experimental.pallas import tpu as pltpu
from jax.experimental.pallas import tpu_sc as plsc

_B, _C, _N, _L = 2, 3, 4, 64
_BCN = _B * _C * _N
_SCB = 4
_NQ = 8
_QW = _L // _NQ
_SLAB = 4
_NSLAB = _QW // _SLAB
_F32 = jnp.float32


def _sc_body(l_hbm, m_hbm, o_dp, o_hp, o_wp,
             lbuf, mbuf, s_dp, s_hp, s_wp):
    wid = lax.axis_index("s") * 2 + lax.axis_index("c")
    box = wid // _NQ
    quarter = wid % _NQ
    b = box // (_C * _N)
    c = (box // _N) % _C
    n = box % _N
    zero = jnp.zeros((16,), _F32)

    for dc in range(4):
        s_dp[pl.ds(dc * 16, 16)] = zero

    def zero_h(h, _):
        s_hp[pl.ds(h * 16, 16)] = zero
        return 0

    lax.fori_loop(0, _L, zero_h, 0)

    def slab_body(slab, _):
        w0 = pl.multiple_of(quarter * _QW + slab * _SLAB, _SLAB)
        pltpu.sync_copy(l_hbm.at[b, c, pl.ds(w0, _SLAB)], lbuf)
        pltpu.sync_copy(m_hbm.at[b, c, n, pl.ds(w0, _SLAB)], mbuf)

        def h_body(h, carry):
            acc = list(carry)
            hacc = [zero] * 4
            for w in range(_SLAB):
                for dc in range(4):
                    lv = lbuf[w, h, pl.ds(dc * 16, 16)]
                    mv = mbuf[w, h, pl.ds(dc * 16, 16)]
                    pv = lv * mv
                    acc[dc] = acc[dc] + pv
                    hacc[dc] = hacc[dc] + pv
                    acc[4 + w] = acc[4 + w] + pv
            hsum = (hacc[0] + hacc[1]) + (hacc[2] + hacc[3])
            plsc.addupdate(s_hp.at[pl.ds(h * 16, 16)], hsum)
            return tuple(acc)

        acc = lax.fori_loop(0, _L, h_body, (zero,) * (4 + _SLAB))
        for dc in range(4):
            plsc.addupdate(s_dp.at[pl.ds(dc * 16, 16)], acc[dc])
        for w in range(_SLAB):
            s_wp[pl.ds((slab * _SLAB + w) * 16, 16)] = acc[4 + w]
        return 0

    lax.fori_loop(0, _NSLAB, slab_body, 0)

    pltpu.sync_copy(s_dp, o_dp.at[wid])
    pltpu.sync_copy(s_hp, o_hp.at[wid])
    pltpu.sync_copy(s_wp, o_wp.at[wid])


_sc_profiles = functools.partial(
    pl.kernel,
    out_type=(
        jax.ShapeDtypeStruct((_SCB * _NQ, 64), _F32),
        jax.ShapeDtypeStruct((_SCB * _NQ, 1024), _F32),
        jax.ShapeDtypeStruct((_SCB * _NQ, _QW * 16), _F32),
    ),
    mesh=plsc.VectorSubcoreMesh(core_axis_name="c", subcore_axis_name="s"),
    scratch_types=[
        pltpu.VMEM((_SLAB, _L, _L), _F32),
        pltpu.VMEM((_SLAB, _L, _L), _F32),
        pltpu.VMEM((64,), _F32),
        pltpu.VMEM((1024,), _F32),
        pltpu.VMEM((_QW * 16,), _F32),
    ],
    compiler_params=pltpu.CompilerParams(use_tc_tiling_on_sc=True, skip_device_barrier=True),
)(_sc_body)


_TCB = _BCN - _SCB
_TCG = _TCB // _N


def _tc_body(l_ref, m_ref, dp_ref, hp_ref, wp_ref):
    lg = l_ref[0, 0]
    for n in range(_N):
        p = lg * m_ref[0, 0, n]
        a = p.sum(axis=0)
        dp_ref[0, n] = a.sum(axis=0)
        hp_ref[0, n] = a.sum(axis=1)
        wp_ref[0, n] = p.sum(axis=(1, 2))


def _tc_index_l(g):
    return (g + _SCB // _N) // _C, (g + _SCB // _N) % _C, 0, 0, 0


def _tc_index_m(g):
    return (g + _SCB // _N) // _C, (g + _SCB // _N) % _C, 0, 0, 0, 0


_tc_profiles = functools.partial(
    pl.pallas_call,
    grid=(_TCG,),
    in_specs=[
        pl.BlockSpec((1, 1, _L, _L, _L), _tc_index_l),
        pl.BlockSpec((1, 1, _N, _L, _L, _L), _tc_index_m),
    ],
    out_specs=[
        pl.BlockSpec((1, _N, _L), lambda g: (g, 0, 0)),
        pl.BlockSpec((1, _N, _L), lambda g: (g, 0, 0)),
        pl.BlockSpec((1, _N, _L), lambda g: (g, 0, 0)),
    ],
    out_shape=[
        jax.ShapeDtypeStruct((_TCG, _N, _L), _F32),
        jax.ShapeDtypeStruct((_TCG, _N, _L), _F32),
        jax.ShapeDtypeStruct((_TCG, _N, _L), _F32),
    ],
    compiler_params=pltpu.CompilerParams(
        dimension_semantics=("arbitrary",),
    ),
)(_tc_body)


def _axis_err(s):
    m = (s > 0).astype(_F32)
    cnt = m.sum(-1)
    valid = cnt > 0
    mean = jnp.where(valid, s.sum(-1) / jnp.maximum(cnt, 1.0), 0.0)
    return jnp.maximum(jnp.where(valid, 1.0 - mean, 0.0), 0.0)


def _lane_sel(rows, cols):
    r = lax.broadcasted_iota(jnp.int32, (rows, cols), 0)
    c = lax.broadcasted_iota(jnp.int32, (rows, cols), 1)
    return (r // 16 == c).astype(_F32)


def _win_errs(mat):
    tot = _axis_err(mat[:, 0:8])
    for win in range(1, 8):
        tot = tot + _axis_err(mat[:, win * 8:(win + 1) * 8])
    return tot


def _epi_body(sdp, shp, swp, tdp, thp, twp, out):
    hi = jax.lax.Precision.HIGHEST
    sdp3 = sdp[...].reshape(_SCB, _NQ, 64).sum(axis=1)
    shp_h = jnp.dot(shp[...].reshape(_SCB, _NQ, 1024).sum(axis=1),
                    _lane_sel(1024, 64), precision=hi)
    swp_w = jnp.dot(swp[...], _lane_sel(_QW * 16, _QW), precision=hi)
    e_w = _axis_err(swp_w).reshape(_SCB, _NQ).sum(axis=1)
    tot_sc = (_win_errs(sdp3) + _win_errs(shp_h) + e_w) * 8.0
    tdp2 = tdp[...].reshape(_TCB, 64)
    thp2 = thp[...].reshape(_TCB, 64)
    twp2 = twp[...].reshape(_TCB, 64)
    tot_tc = (_win_errs(tdp2) + _win_errs(thp2) + _win_errs(twp2)) * 8.0
    out[0, 0] = jnp.sum(tot_sc * tot_sc) + jnp.sum(tot_tc * tot_tc)


def kernel(logits, box_masks):
    sdp, shp, swp = _sc_profiles(logits, box_masks)
    tdp, thp, twp = _tc_profiles(logits, box_masks)
    loss = pl.pallas_call(
        _epi_body,
        out_shape=jax.ShapeDtypeStruct((1, 1), _F32),
        out_specs=pl.BlockSpec(memory_space=pltpu.SMEM),
    )(sdp, shp, swp, tdp, thp, twp)
    return loss[0, 0]
